# baseline (speedup 1.0000x reference)
import functools

import jax
import jax.numpy as jnp
from jax import lax
from jax.experimental import pallas as pl
from jax.experimental.pallas import tpu as pltpu
from jax.experimental.pallas import tpu_sc as plsc

N = 10000
D = 128
H = 1024
E = 640000
NC = 2
NS = 16
TD = 64
TDD = 80
CHUNK = 128
EPW = 40960
RPW = EPW // CHUNK
SUP = 16
NBUF = 4
LA = 2
NSUP = RPW // SUP
NACC = 10112
ZR = NACC // NS
ZHALF = ZR // 2
OPS = 624

_mesh = plsc.VectorSubcoreMesh(core_axis_name="c", subcore_axis_name="s")


def _stack_halves(x):
    return jnp.concatenate([x[:, :TD], x[:, TD:]], axis=0)


def _stack_halves_deg(x):
    ones = jnp.ones((N, TDD - TD), jnp.float32)
    top = jnp.concatenate([x[:, :TD], ones], axis=1)
    bot = jnp.concatenate([x[:, TD:], ones], axis=1)
    return jnp.concatenate([top, bot], axis=0)


def _prep_edges(ei, ew):
    eps = E // NS
    padn = EPW - eps
    src = ei[0].astype(jnp.int32).reshape(NS, eps)
    dst = ei[1].astype(jnp.int32).reshape(NS, eps)
    dummy = N + (jnp.arange(padn, dtype=jnp.int32) % NS)
    src = jnp.concatenate([src, jnp.zeros((NS, padn), jnp.int32)], axis=1)
    dst = jnp.concatenate([dst, jnp.tile(dummy[None, :], (NS, 1))], axis=1)
    src = src.reshape(NS * RPW, CHUNK)
    dst = dst.reshape(NS * RPW, CHUNK)
    ew2 = None
    if ew is not None:
        ew2 = jnp.concatenate(
            [ew.reshape(NS, eps), jnp.zeros((NS, padn), jnp.float32)], axis=1
        ).reshape(NS * RPW, CHUNK)
    return src, dst, ew2


@functools.cache
def _seg_call(weighted: bool, w: int):
    n_in = 4 if weighted else 3

    def body(*refs):
        ins = refs[:n_in]
        out = refs[n_in]
        (src_v, dst_v, ew_v, zb, acc) = refs[n_in + 1:n_in + 6]
        rows = refs[n_in + 6:n_in + 6 + NBUF]
        gsems = refs[n_in + 6 + NBUF:n_in + 6 + 2 * NBUF]
        ssems = refs[n_in + 6 + 2 * NBUF:n_in + 6 + 3 * NBUF]
        table, src2, dst2 = ins[0], ins[1], ins[2]
        ew2 = ins[3] if weighted else None

        c = lax.axis_index("c")
        s = lax.axis_index("s")

        z16 = jnp.zeros((16,), jnp.float32)

        def _zb(r, carry):
            for g in range(w // 16):
                zb[r, pl.ds(g * 16, 16)] = z16
            return carry

        lax.fori_loop(0, ZHALF, _zb, 0)

        z0 = s * ZR
        pltpu.sync_copy(zb, acc.at[pl.ds(z0, ZHALF)])
        pltpu.sync_copy(zb, acc.at[pl.ds(z0 + ZHALF, ZHALF)])
        plsc.subcore_barrier()

        base = s * RPW
        cn16 = jnp.full((16,), c * N, jnp.int32)

        def _mul_rows(rv, j):
            def _mul(gg, cc2):
                wv = ew_v[j, pl.ds(gg * 16, 16)]
                for l in range(16):
                    wvec = jnp.full((16,), wv[l])
                    row = gg * 16 + l
                    for g in range(TD // 16):
                        rv[row, pl.ds(g * 16, 16)] = rv[row, pl.ds(g * 16, 16)] * wvec
                return cc2

            lax.fori_loop(0, CHUNK // 16, _mul, 0)

        def _super(t, carry):
            r0 = base + t * SUP
            pltpu.sync_copy(src2.at[pl.ds(r0, SUP)], src_v)
            pltpu.sync_copy(dst2.at[pl.ds(r0, SUP)], dst_v)
            if weighted:
                pltpu.sync_copy(ew2.at[pl.ds(r0, SUP)], ew_v)

            def _off(r, carry2):
                for g in range(CHUNK // 16):
                    src_v[r, pl.ds(g * 16, 16)] = src_v[r, pl.ds(g * 16, 16)] + cn16
                return carry2

            lax.fori_loop(0, SUP, _off, 0)

            gd = [None] * NBUF
            sd = [None] * NBUF
            for j in range(LA):
                gd[j] = pltpu.async_copy(table.at[src_v.at[j]], rows[j], gsems[j])
            for j in range(SUP):
                b = j % NBUF
                if j + LA < SUP:
                    b2 = (j + LA) % NBUF
                    if sd[b2] is not None:
                        sd[b2].wait()
                        sd[b2] = None
                    gd[b2] = pltpu.async_copy(table.at[src_v.at[j + LA]], rows[b2], gsems[b2])
                gd[b].wait()
                if weighted:
                    _mul_rows(rows[b], j)
                sd[b] = pltpu.async_copy(rows[b], acc.at[dst_v.at[j]], ssems[b], add=True)

            for b in range(NBUF):
                if sd[b] is not None:
                    sd[b].wait()

            return carry

        lax.fori_loop(0, NSUP, _super, 0)
        plsc.subcore_barrier()

        o0 = s * OPS
        pltpu.sync_copy(acc.at[pl.ds(o0, OPS)], out.at[pl.ds(c * N + o0, OPS)])

        tail = NS * OPS

        @pl.when(s < 2)
        def _():
            t0 = tail + s * 8
            pltpu.sync_copy(acc.at[pl.ds(t0, 8)], out.at[pl.ds(c * N + t0, 8)])

    return pl.kernel(
        body,
        out_type=jax.ShapeDtypeStruct((NC * N, w), jnp.float32),
        mesh=_mesh,
        compiler_params=pltpu.CompilerParams(use_tc_tiling_on_sc=False),
        scratch_types=[
            pltpu.VMEM((SUP, CHUNK), jnp.int32),
            pltpu.VMEM((SUP, CHUNK), jnp.int32),
            pltpu.VMEM((SUP, CHUNK), jnp.float32),
            pltpu.VMEM((ZHALF, w), jnp.float32),
            pltpu.VMEM_SHARED((NACC, w), jnp.float32),
        ] + [pltpu.VMEM((CHUNK, w), jnp.float32)] * NBUF
          + [pltpu.SemaphoreType.DMA] * (2 * NBUF),
    )


BN = 1000


def _dense_body(x, p0, p1, ws0, wn0, b0, ws1, wn1, b1, y, sout):
    degv = jnp.maximum(p0[:, TD:TD + 1], 1.0)
    a0 = jnp.concatenate([p0[:, :TD], p1[:, :TD]], axis=1) / degv
    h = jnp.dot(x[...], ws0[...], preferred_element_type=jnp.float32)
    h += jnp.dot(a0, wn0[...], preferred_element_type=jnp.float32)
    h = jnp.maximum(h + b0[...], 0.0)
    y[...] = jnp.dot(h, wn1[...], preferred_element_type=jnp.float32)
    sout[...] = jnp.dot(h, ws1[...], preferred_element_type=jnp.float32) + b1[...]


def _dense(x, p, ws0, wn0, b0, ws1, wn1, b1):
    hd = ws0.shape[1]
    nb = N // BN
    row = lambda i: (i, 0)
    row2 = lambda i: (i + nb, 0)
    full = lambda: (lambda i: (0, 0))
    return pl.pallas_call(
        _dense_body,
        grid=(nb,),
        in_specs=[
            pl.BlockSpec((BN, D), row),
            pl.BlockSpec((BN, TDD), row),
            pl.BlockSpec((BN, TDD), row2),
            pl.BlockSpec((D, hd), full()),
            pl.BlockSpec((D, hd), full()),
            pl.BlockSpec((1, hd), full()),
            pl.BlockSpec((hd, D), full()),
            pl.BlockSpec((hd, D), full()),
            pl.BlockSpec((1, D), full()),
        ],
        out_specs=[pl.BlockSpec((BN, D), row), pl.BlockSpec((BN, D), row)],
        out_shape=[jax.ShapeDtypeStruct((N, D), jnp.float32)] * 2,
    )(x, p, p, ws0, wn0, b0, ws1, wn1, b1)


def _fin(sref, p0, p1, d):
    degv = jnp.maximum(d[:, TD:TD + 1], 1.0)
    return sref[...] + jnp.concatenate([p0[:, :TD], p1[:, :TD]], axis=1) / degv


def _att_stats_body(pr_s, pr_p0, pr_p1, pr_d,
                    cc_s, cc_p0, cc_p1, cc_d,
                    ap_s, ap_p0, ap_p1, ap_d,
                    att_w, att_b, pr_o, cc_o, w_o):
    pr = _fin(pr_s, pr_p0, pr_p1, pr_d)
    cc = _fin(cc_s, cc_p0, cc_p1, cc_d)
    ap = _fin(ap_s, ap_p0, ap_p1, ap_d)
    pr_o[...] = pr
    cc_o[...] = cc
    proj0 = jnp.tanh(jnp.dot(pr, att_w[...], preferred_element_type=jnp.float32) + att_b[...])
    proj1 = jnp.tanh(jnp.dot(cc, att_w[...], preferred_element_type=jnp.float32) + att_b[...])
    w0 = jnp.sum(ap * proj0)
    w1 = jnp.sum(ap * proj1)
    i = pl.program_id(0)

    @pl.when(i == 0)
    def _():
        w_o[...] = jnp.zeros((8, 128), jnp.float32)

    r = lax.broadcasted_iota(jnp.int32, (8, 128), 0)
    cix = lax.broadcasted_iota(jnp.int32, (8, 128), 1)
    upd = jnp.where((r == 0) & (cix == 0), w0, 0.0) + jnp.where((r == 0) & (cix == 1), w1, 0.0)
    w_o[...] = w_o[...] + upd


def _att_stats(pr_args, cc_args, ap_args, att_w, att_b):
    nb = N // BN
    row = lambda i: (i, 0)
    row2 = lambda i: (i + nb, 0)
    branch_specs = [
        pl.BlockSpec((BN, D), row),
        pl.BlockSpec((BN, TD), row),
        pl.BlockSpec((BN, TD), row2),
        pl.BlockSpec((BN, TDD), row),
    ]
    return pl.pallas_call(
        _att_stats_body,
        grid=(nb,),
        in_specs=branch_specs * 3 + [
            pl.BlockSpec((D, D), lambda i: (0, 0)),
            pl.BlockSpec((1, D), lambda i: (0, 0)),
        ],
        out_specs=[
            pl.BlockSpec((BN, D), row),
            pl.BlockSpec((BN, D), row),
            pl.BlockSpec((8, 128), lambda i: (0, 0)),
        ],
        out_shape=[
            jax.ShapeDtypeStruct((N, D), jnp.float32),
            jax.ShapeDtypeStruct((N, D), jnp.float32),
            jax.ShapeDtypeStruct((8, 128), jnp.float32),
        ],
    )(*pr_args, *cc_args, *ap_args, att_w, att_b)


def _att_combine_body(pr, cc, w, out):
    w0 = w[0, 0] / N
    w1 = w[0, 1] / N
    m = jnp.maximum(w0, w1)
    e0 = jnp.exp(w0 - m)
    e1 = jnp.exp(w1 - m)
    inv = 1.0 / (e0 + e1)
    out[...] = pr[...] * (e0 * inv) + cc[...] * (e1 * inv)


def _att_combine(pr, cc, w):
    nb = N // BN
    row = lambda i: (i, 0)
    return pl.pallas_call(
        _att_combine_body,
        grid=(nb,),
        in_specs=[
            pl.BlockSpec((BN, D), row),
            pl.BlockSpec((BN, D), row),
            pl.BlockSpec((8, 128), lambda i: (0, 0)),
        ],
        out_specs=pl.BlockSpec((BN, D), row),
        out_shape=jax.ShapeDtypeStruct((N, D), jnp.float32),
    )(pr, cc, w)


def kernel(g_edge_index, pr_edge_index, cc_edge_index, sps_edge_index, pr_ew, cc_ew, k_emb,
           pr0_Wself, pr0_Wneigh, pr0_b, pr1_Wself, pr1_Wneigh, pr1_b,
           cc0_Wself, cc0_Wneigh, cc0_b, cc1_Wself, cc1_Wneigh, cc1_b,
           ap0_Wself, ap0_Wneigh, ap0_b, ap1_Wself, ap1_Wneigh, ap1_b,
           att_W, att_b):
    del g_edge_index
    pr_src, pr_dst, pr_w2 = _prep_edges(pr_edge_index, pr_ew)
    cc_src, cc_dst, cc_w2 = _prep_edges(cc_edge_index, cc_ew)
    sp_src, sp_dst, _ = _prep_edges(sps_edge_index, None)
    k_st = _stack_halves_deg(k_emb)

    seg_w0 = _seg_call(True, TDD)
    seg_w1 = _seg_call(True, TD)
    seg_u0 = _seg_call(False, TDD)
    seg_u1 = _seg_call(False, TD)

    p_pr = seg_w0(k_st, pr_src, pr_dst, pr_w2)
    p_cc = seg_w0(k_st, cc_src, cc_dst, cc_w2)
    p_ap = seg_u0(k_st, sp_src, sp_dst)

    y_pr, s_pr = _dense(k_emb, p_pr, pr0_Wself, pr0_Wneigh,
                        pr0_b.reshape(1, H), pr1_Wself, pr1_Wneigh, pr1_b.reshape(1, D))
    y_cc, s_cc = _dense(k_emb, p_cc, cc0_Wself, cc0_Wneigh,
                        cc0_b.reshape(1, H), cc1_Wself, cc1_Wneigh, cc1_b.reshape(1, D))
    y_ap, s_ap = _dense(k_emb, p_ap, ap0_Wself, ap0_Wneigh,
                        ap0_b.reshape(1, D), ap1_Wself, ap1_Wneigh, ap1_b.reshape(1, D))

    p1_pr = seg_w1(_stack_halves(y_pr), pr_src, pr_dst, pr_w2)
    p1_cc = seg_w1(_stack_halves(y_cc), cc_src, cc_dst, cc_w2)
    p1_ap = seg_u1(_stack_halves(y_ap), sp_src, sp_dst)

    pr_f, cc_f, w = _att_stats(
        (s_pr, p1_pr, p1_pr, p_pr),
        (s_cc, p1_cc, p1_cc, p_cc),
        (s_ap, p1_ap, p1_ap, p_ap),
        att_W, att_b.reshape(1, D))
    return _att_combine(pr_f, cc_f, w)


# speedup vs baseline: 1.2534x; 1.2534x over previous
import functools

import jax
import jax.numpy as jnp
from jax import lax
from jax.experimental import pallas as pl
from jax.experimental.pallas import tpu as pltpu
from jax.experimental.pallas import tpu_sc as plsc

N = 10000
D = 128
H = 1024
E = 640000
NC = 2
NS = 16
TD = 64
CHUNK = 128
EPW = 40960
RPW = EPW // CHUNK
SUP = 16
NBUF = 4
LA = 2
NSUP = RPW // SUP
NACC = 10112
ZR = NACC // NS
ZHALF = ZR // 2
OPS = 624
DEGW = 16

_mesh = plsc.VectorSubcoreMesh(core_axis_name="c", subcore_axis_name="s")


def _stack_halves(x):
    return jnp.concatenate([x[:, :TD], x[:, TD:]], axis=0)


def _prep_edges(ei, ew):
    eps = E // NS
    padn = EPW - eps
    src = ei[0].astype(jnp.int32).reshape(NS, eps)
    dst = ei[1].astype(jnp.int32).reshape(NS, eps)
    dummy = N + (jnp.arange(padn, dtype=jnp.int32) % NS)
    src = jnp.concatenate([src, jnp.zeros((NS, padn), jnp.int32)], axis=1)
    dst = jnp.concatenate([dst, jnp.tile(dummy[None, :], (NS, 1))], axis=1)
    src = src.reshape(NS * RPW, CHUNK)
    dst = dst.reshape(NS * RPW, CHUNK)
    ew2 = None
    if ew is not None:
        ew2 = jnp.concatenate(
            [ew.reshape(NS, eps), jnp.zeros((NS, padn), jnp.float32)], axis=1
        ).reshape(NS * RPW, CHUNK)
    return src, dst, ew2


@functools.cache
def _seg_call(weighted: bool, with_deg: bool):
    n_in = 4 if weighted else 3
    out_type = [jax.ShapeDtypeStruct((NC * N, TD), jnp.float32)]
    if with_deg:
        out_type.append(jax.ShapeDtypeStruct((NC * N, DEGW), jnp.float32))
    n_out = len(out_type)

    def body(*refs):
        ins = refs[:n_in]
        outs = refs[n_in:n_in + n_out]
        (src_v, dst_v, ew_v, zb, zbd, ones_v, acc, dacc) = refs[n_in + n_out:n_in + n_out + 8]
        rows = refs[n_in + n_out + 8:n_in + n_out + 8 + NBUF]
        gsems = refs[n_in + n_out + 8 + NBUF:n_in + n_out + 8 + 2 * NBUF]
        ssems = refs[n_in + n_out + 8 + 2 * NBUF:n_in + n_out + 8 + 3 * NBUF]
        dsem = refs[-1]
        table, src2, dst2 = ins[0], ins[1], ins[2]
        ew2 = ins[3] if weighted else None
        out = outs[0]
        dout = outs[1] if with_deg else None

        c = lax.axis_index("c")
        s = lax.axis_index("s")

        z16 = jnp.zeros((16,), jnp.float32)

        def _zb(r, carry):
            for g in range(TD // 16):
                zb[r, pl.ds(g * 16, 16)] = z16
            zbd[r, pl.ds(0, 16)] = z16
            return carry

        lax.fori_loop(0, ZHALF, _zb, 0)
        if with_deg:
            o16 = jnp.full((16,), 1.0, jnp.float32)

            def _ob(r, carry):
                ones_v[r, pl.ds(0, 16)] = o16
                return carry

            lax.fori_loop(0, CHUNK, _ob, 0)

        z0 = s * ZR
        pltpu.sync_copy(zb, acc.at[pl.ds(z0, ZHALF)])
        pltpu.sync_copy(zb, acc.at[pl.ds(z0 + ZHALF, ZHALF)])
        if with_deg:
            pltpu.sync_copy(zbd, dacc.at[pl.ds(z0, ZHALF)])
            pltpu.sync_copy(zbd, dacc.at[pl.ds(z0 + ZHALF, ZHALF)])

        plsc.subcore_barrier()

        base = s * RPW
        cn16 = jnp.full((16,), c * N, jnp.int32)

        def _mul_rows(rv, j):
            def _mul(gg, cc2):
                wv = ew_v[j, pl.ds(gg * 16, 16)]
                for l in range(16):
                    w = jnp.full((16,), wv[l])
                    row = gg * 16 + l
                    for g in range(TD // 16):
                        rv[row, pl.ds(g * 16, 16)] = rv[row, pl.ds(g * 16, 16)] * w
                return cc2

            lax.fori_loop(0, CHUNK // 16, _mul, 0)

        def _super(t, carry):
            r0 = base + t * SUP
            pltpu.sync_copy(src2.at[pl.ds(r0, SUP)], src_v)
            pltpu.sync_copy(dst2.at[pl.ds(r0, SUP)], dst_v)
            if weighted:
                pltpu.sync_copy(ew2.at[pl.ds(r0, SUP)], ew_v)

            def _off(r, carry2):
                for g in range(CHUNK // 16):
                    src_v[r, pl.ds(g * 16, 16)] = src_v[r, pl.ds(g * 16, 16)] + cn16
                return carry2

            lax.fori_loop(0, SUP, _off, 0)

            do_deg = lax.rem(t, 2) == c

            gd = [None] * NBUF
            sd = [None] * NBUF
            degs = []
            for j in range(LA):
                gd[j] = pltpu.async_copy(table.at[src_v.at[j]], rows[j], gsems[j])
            for j in range(SUP):
                b = j % NBUF
                if j + LA < SUP:
                    b2 = (j + LA) % NBUF
                    if sd[b2] is not None:
                        sd[b2].wait()
                        sd[b2] = None
                    gd[b2] = pltpu.async_copy(table.at[src_v.at[j + LA]], rows[b2], gsems[b2])
                gd[b].wait()
                if weighted:
                    _mul_rows(rows[b], j)
                sd[b] = pltpu.async_copy(rows[b], acc.at[dst_v.at[j]], ssems[b], add=True)
                if with_deg:

                    @pl.when(do_deg)
                    def _():
                        degs.append(pltpu.async_copy(ones_v, dacc.at[dst_v.at[j]], dsem, add=True))

            for b in range(NBUF):
                if sd[b] is not None:
                    sd[b].wait()
            if with_deg:

                @pl.when(do_deg)
                def _():
                    for dd in degs:
                        dd.wait()

            return carry

        lax.fori_loop(0, NSUP, _super, 0)
        plsc.subcore_barrier()

        o0 = s * OPS
        pltpu.sync_copy(acc.at[pl.ds(o0, OPS)], out.at[pl.ds(c * N + o0, OPS)])
        if with_deg:
            pltpu.sync_copy(dacc.at[pl.ds(o0, OPS)], dout.at[pl.ds(c * N + o0, OPS)])

        tail = NS * OPS

        @pl.when(s < 2)
        def _():
            t0 = tail + s * 8
            pltpu.sync_copy(acc.at[pl.ds(t0, 8)], out.at[pl.ds(c * N + t0, 8)])
            if with_deg:
                pltpu.sync_copy(dacc.at[pl.ds(t0, 8)], dout.at[pl.ds(c * N + t0, 8)])

    return pl.kernel(
        body,
        out_type=out_type if n_out > 1 else out_type[0],
        mesh=_mesh,
        compiler_params=pltpu.CompilerParams(use_tc_tiling_on_sc=False),
        scratch_types=[
            pltpu.VMEM((SUP, CHUNK), jnp.int32),
            pltpu.VMEM((SUP, CHUNK), jnp.int32),
            pltpu.VMEM((SUP, CHUNK), jnp.float32),
            pltpu.VMEM((ZHALF, TD), jnp.float32),
            pltpu.VMEM((ZHALF, DEGW), jnp.float32),
            pltpu.VMEM((CHUNK, DEGW), jnp.float32),
            pltpu.VMEM_SHARED((NACC, TD), jnp.float32),
            pltpu.VMEM_SHARED((NACC, DEGW), jnp.float32),
        ] + [pltpu.VMEM((CHUNK, TD), jnp.float32)] * NBUF
          + [pltpu.SemaphoreType.DMA] * (2 * NBUF + 1),
    )


BN = 1000


def _dense_body(x, p0, p1, d0, d1, ws0, wn0, b0, ws1, wn1, b1, y, sout):
    degv = jnp.maximum(d0[:, 0:1] + d1[:, 0:1], 1.0)
    a0 = jnp.concatenate([p0[...], p1[...]], axis=1) / degv
    h = jnp.dot(x[...], ws0[...], preferred_element_type=jnp.float32)
    h += jnp.dot(a0, wn0[...], preferred_element_type=jnp.float32)
    h = jnp.maximum(h + b0[...], 0.0)
    y[...] = jnp.dot(h, wn1[...], preferred_element_type=jnp.float32)
    sout[...] = jnp.dot(h, ws1[...], preferred_element_type=jnp.float32) + b1[...]


def _dense(x, p, deg, ws0, wn0, b0, ws1, wn1, b1):
    hd = ws0.shape[1]
    nb = N // BN
    row = lambda i: (i, 0)
    row2 = lambda i: (i + nb, 0)
    full = lambda: (lambda i: (0, 0))
    return pl.pallas_call(
        _dense_body,
        grid=(nb,),
        in_specs=[
            pl.BlockSpec((BN, D), row),
            pl.BlockSpec((BN, TD), row),
            pl.BlockSpec((BN, TD), row2),
            pl.BlockSpec((BN, DEGW), row),
            pl.BlockSpec((BN, DEGW), row2),
            pl.BlockSpec((D, hd), full()),
            pl.BlockSpec((D, hd), full()),
            pl.BlockSpec((1, hd), full()),
            pl.BlockSpec((hd, D), full()),
            pl.BlockSpec((hd, D), full()),
            pl.BlockSpec((1, D), full()),
        ],
        out_specs=[pl.BlockSpec((BN, D), row), pl.BlockSpec((BN, D), row)],
        out_shape=[jax.ShapeDtypeStruct((N, D), jnp.float32)] * 2,
    )(x, p, p, deg, deg, ws0, wn0, b0, ws1, wn1, b1)


def _fin(sref, p0, p1, d0, d1):
    degv = jnp.maximum(d0[:, 0:1] + d1[:, 0:1], 1.0)
    return sref[...] + jnp.concatenate([p0[...], p1[...]], axis=1) / degv


def _att_stats_body(pr_s, pr_p0, pr_p1, pr_d0, pr_d1,
                    cc_s, cc_p0, cc_p1, cc_d0, cc_d1,
                    ap_s, ap_p0, ap_p1, ap_d0, ap_d1,
                    att_w, att_b, pr_o, cc_o, w_o):
    pr = _fin(pr_s, pr_p0, pr_p1, pr_d0, pr_d1)
    cc = _fin(cc_s, cc_p0, cc_p1, cc_d0, cc_d1)
    ap = _fin(ap_s, ap_p0, ap_p1, ap_d0, ap_d1)
    pr_o[...] = pr
    cc_o[...] = cc
    proj0 = jnp.tanh(jnp.dot(pr, att_w[...], preferred_element_type=jnp.float32) + att_b[...])
    proj1 = jnp.tanh(jnp.dot(cc, att_w[...], preferred_element_type=jnp.float32) + att_b[...])
    w0 = jnp.sum(ap * proj0)
    w1 = jnp.sum(ap * proj1)
    i = pl.program_id(0)

    @pl.when(i == 0)
    def _():
        w_o[...] = jnp.zeros((8, 128), jnp.float32)

    r = lax.broadcasted_iota(jnp.int32, (8, 128), 0)
    cix = lax.broadcasted_iota(jnp.int32, (8, 128), 1)
    upd = jnp.where((r == 0) & (cix == 0), w0, 0.0) + jnp.where((r == 0) & (cix == 1), w1, 0.0)
    w_o[...] = w_o[...] + upd


def _att_stats(pr_args, cc_args, ap_args, att_w, att_b):
    nb = N // BN
    row = lambda i: (i, 0)
    row2 = lambda i: (i + nb, 0)
    branch_specs = [
        pl.BlockSpec((BN, D), row),
        pl.BlockSpec((BN, TD), row),
        pl.BlockSpec((BN, TD), row2),
        pl.BlockSpec((BN, DEGW), row),
        pl.BlockSpec((BN, DEGW), row2),
    ]
    return pl.pallas_call(
        _att_stats_body,
        grid=(nb,),
        in_specs=branch_specs * 3 + [
            pl.BlockSpec((D, D), lambda i: (0, 0)),
            pl.BlockSpec((1, D), lambda i: (0, 0)),
        ],
        out_specs=[
            pl.BlockSpec((BN, D), row),
            pl.BlockSpec((BN, D), row),
            pl.BlockSpec((8, 128), lambda i: (0, 0)),
        ],
        out_shape=[
            jax.ShapeDtypeStruct((N, D), jnp.float32),
            jax.ShapeDtypeStruct((N, D), jnp.float32),
            jax.ShapeDtypeStruct((8, 128), jnp.float32),
        ],
    )(*pr_args, *cc_args, *ap_args, att_w, att_b)


def _att_combine_body(pr, cc, w, out):
    w0 = w[0, 0] / N
    w1 = w[0, 1] / N
    m = jnp.maximum(w0, w1)
    e0 = jnp.exp(w0 - m)
    e1 = jnp.exp(w1 - m)
    inv = 1.0 / (e0 + e1)
    out[...] = pr[...] * (e0 * inv) + cc[...] * (e1 * inv)


def _att_combine(pr, cc, w):
    nb = N // BN
    row = lambda i: (i, 0)
    return pl.pallas_call(
        _att_combine_body,
        grid=(nb,),
        in_specs=[
            pl.BlockSpec((BN, D), row),
            pl.BlockSpec((BN, D), row),
            pl.BlockSpec((8, 128), lambda i: (0, 0)),
        ],
        out_specs=pl.BlockSpec((BN, D), row),
        out_shape=jax.ShapeDtypeStruct((N, D), jnp.float32),
    )(pr, cc, w)


def kernel(g_edge_index, pr_edge_index, cc_edge_index, sps_edge_index, pr_ew, cc_ew, k_emb,
           pr0_Wself, pr0_Wneigh, pr0_b, pr1_Wself, pr1_Wneigh, pr1_b,
           cc0_Wself, cc0_Wneigh, cc0_b, cc1_Wself, cc1_Wneigh, cc1_b,
           ap0_Wself, ap0_Wneigh, ap0_b, ap1_Wself, ap1_Wneigh, ap1_b,
           att_W, att_b):
    del g_edge_index
    pr_src, pr_dst, pr_w2 = _prep_edges(pr_edge_index, pr_ew)
    cc_src, cc_dst, cc_w2 = _prep_edges(cc_edge_index, cc_ew)
    sp_src, sp_dst, _ = _prep_edges(sps_edge_index, None)
    k_st = _stack_halves(k_emb)

    seg_w0 = _seg_call(True, True)
    seg_w1 = _seg_call(True, False)
    seg_u0 = _seg_call(False, True)
    seg_u1 = _seg_call(False, False)

    p_pr, deg_pr = seg_w0(k_st, pr_src, pr_dst, pr_w2)
    p_cc, deg_cc = seg_w0(k_st, cc_src, cc_dst, cc_w2)
    p_ap, deg_ap = seg_u0(k_st, sp_src, sp_dst)

    y_pr, s_pr = _dense(k_emb, p_pr, deg_pr, pr0_Wself, pr0_Wneigh,
                        pr0_b.reshape(1, H), pr1_Wself, pr1_Wneigh, pr1_b.reshape(1, D))
    y_cc, s_cc = _dense(k_emb, p_cc, deg_cc, cc0_Wself, cc0_Wneigh,
                        cc0_b.reshape(1, H), cc1_Wself, cc1_Wneigh, cc1_b.reshape(1, D))
    y_ap, s_ap = _dense(k_emb, p_ap, deg_ap, ap0_Wself, ap0_Wneigh,
                        ap0_b.reshape(1, D), ap1_Wself, ap1_Wneigh, ap1_b.reshape(1, D))

    p1_pr = seg_w1(_stack_halves(y_pr), pr_src, pr_dst, pr_w2)
    p1_cc = seg_w1(_stack_halves(y_cc), cc_src, cc_dst, cc_w2)
    p1_ap = seg_u1(_stack_halves(y_ap), sp_src, sp_dst)

    pr_f, cc_f, w = _att_stats(
        (s_pr, p1_pr, p1_pr, deg_pr, deg_pr),
        (s_cc, p1_cc, p1_cc, deg_cc, deg_cc),
        (s_ap, p1_ap, p1_ap, deg_ap, deg_ap),
        att_W, att_b.reshape(1, D))
    return _att_combine(pr_f, cc_f, w)
